# baseline (speedup 1.0000x reference)
import functools

import jax
import jax.numpy as jnp
from jax import lax
from jax.experimental import pallas as pl
from jax.experimental.pallas import tpu as pltpu
from jax.experimental.pallas import tpu_sc as plsc


@functools.lru_cache(maxsize=None)
def _build(B, V, D):
    info = plsc.get_sparse_core_info()
    num_workers = info.num_cores * info.num_subcores
    assert B % (8 * num_workers) == 0
    b_per_w = B // num_workers

    mesh = plsc.VectorSubcoreMesh(core_axis_name="c", subcore_axis_name="s")

    @functools.partial(
        pl.kernel,
        mesh=mesh,
        out_type=jax.ShapeDtypeStruct((B, D), jnp.float32),
        scratch_types=[
            pltpu.VMEM((b_per_w,), jnp.int32),
            pltpu.VMEM((b_per_w, D), jnp.float32),
            pltpu.SemaphoreType.DMA,
            pltpu.SemaphoreType.DMA,
        ],
    )
    def emb_kernel(ids_hbm, table_hbm, out_hbm, idx_v, rows_v, isem, gsem):
        wid = lax.axis_index("s") * info.num_cores + lax.axis_index("c")
        base = wid * b_per_w
        nch = 4
        cw = b_per_w // nch
        iloads = [
            pltpu.async_copy(ids_hbm.at[pl.ds(base + j * cw, cw)],
                             idx_v.at[pl.ds(j * cw, cw)], isem)
            for j in range(nch)
        ]
        gathers = []
        for j in range(nch):
            iloads[j].wait()
            gathers.append(
                pltpu.async_copy(table_hbm.at[idx_v.at[pl.ds(j * cw, cw)]],
                                 rows_v.at[pl.ds(j * cw, cw)], gsem)
            )
        for g in gathers:
            g.wait()
        pltpu.sync_copy(rows_v, out_hbm.at[pl.ds(base, b_per_w)])

    return emb_kernel


def kernel(input_ids, embed_weight):
    B, = input_ids.shape
    V, D = embed_weight.shape
    return _build(B, V, D)(input_ids.astype(jnp.int32), embed_weight)


# speedup vs baseline: 1.0090x; 1.0090x over previous
import functools

import jax
import jax.numpy as jnp
from jax import lax
from jax.experimental import pallas as pl
from jax.experimental.pallas import tpu as pltpu
from jax.experimental.pallas import tpu_sc as plsc


@functools.lru_cache(maxsize=None)
def _build(B, V, D):
    info = plsc.get_sparse_core_info()
    num_workers = info.num_cores * info.num_subcores
    assert B % (8 * num_workers) == 0
    b_per_w = B // num_workers

    mesh = plsc.VectorSubcoreMesh(core_axis_name="c", subcore_axis_name="s")

    @functools.partial(
        pl.kernel,
        mesh=mesh,
        out_type=jax.ShapeDtypeStruct((B, D), jnp.float32),
        scratch_types=[
            pltpu.VMEM((b_per_w,), jnp.int32),
            pltpu.VMEM((b_per_w, D), jnp.float32),
            pltpu.SemaphoreType.DMA,
            pltpu.SemaphoreType.DMA,
        ],
    )
    def emb_kernel(ids_hbm, table_hbm, out_hbm, idx_v, rows_v, isem, gsem):
        wid = lax.axis_index("s") * info.num_cores + lax.axis_index("c")
        base = wid * b_per_w
        half = b_per_w // 2
        i0 = pltpu.async_copy(ids_hbm.at[pl.ds(base, half)],
                              idx_v.at[pl.ds(0, half)], isem)
        i1 = pltpu.async_copy(ids_hbm.at[pl.ds(base + half, half)],
                              idx_v.at[pl.ds(half, half)], isem)
        i0.wait()
        g0 = pltpu.async_copy(table_hbm.at[idx_v.at[pl.ds(0, half)]],
                              rows_v.at[pl.ds(0, half)], gsem)
        i1.wait()
        g1 = pltpu.async_copy(table_hbm.at[idx_v.at[pl.ds(half, half)]],
                              rows_v.at[pl.ds(half, half)], gsem)
        g0.wait()
        g1.wait()
        pltpu.sync_copy(rows_v, out_hbm.at[pl.ds(base, b_per_w)])

    return emb_kernel


def kernel(input_ids, embed_weight):
    B, = input_ids.shape
    V, D = embed_weight.shape
    return _build(B, V, D)(input_ids.astype(jnp.int32), embed_weight)
